# baseline (speedup 1.0000x reference)
import jax
import jax.numpy as jnp
from jax import lax
from jax.experimental import pallas as pl
from jax.experimental.pallas import tpu as pltpu

N_CLS = 1000
PROMPT_LEN = 16
EMBED_DIM = 768
CTX_LEN = 77
SUF_LEN = CTX_LEN - 1 - PROMPT_LEN
SUF_MAIN = 56
OUT_MAIN = 72

C = 10
G = 4
NSTEP = N_CLS // C
NITER = NSTEP // G
NBUF = 2 * G


def _body(pre_hbm, prm_hbm, suf_hbm, out_hbm,
          pre_v, prm_v, suf_v, out_v,
          pre_s, prm_s, suf_s, suf2_s, out_s, out2_s):
    i = pl.program_id(0)

    def in_copies(step):
        slot = lax.rem(step, NBUF)
        c0 = step * C
        cs = pl.ds(c0, C)
        return (
            pltpu.make_async_copy(pre_hbm.at[cs], pre_v.at[slot], pre_s.at[slot]),
            pltpu.make_async_copy(prm_hbm.at[cs], prm_v.at[slot], prm_s.at[slot]),
            pltpu.make_async_copy(suf_hbm.at[cs, pl.ds(0, SUF_MAIN)],
                                  suf_v.at[slot, :, pl.ds(0, SUF_MAIN)],
                                  suf_s.at[slot]),
            pltpu.make_async_copy(suf_hbm.at[cs, pl.ds(SUF_MAIN, SUF_LEN - SUF_MAIN)],
                                  suf_v.at[slot, :, pl.ds(SUF_MAIN, SUF_LEN - SUF_MAIN)],
                                  suf2_s.at[slot]),
        )

    def out_copies(step):
        slot = lax.rem(step, NBUF)
        c0 = step * C
        cs = pl.ds(c0, C)
        return (
            pltpu.make_async_copy(out_v.at[slot, :, pl.ds(0, OUT_MAIN)],
                                  out_hbm.at[cs, pl.ds(0, OUT_MAIN)],
                                  out_s.at[slot]),
            pltpu.make_async_copy(out_v.at[slot, :, pl.ds(OUT_MAIN, CTX_LEN - OUT_MAIN)],
                                  out_hbm.at[cs, pl.ds(OUT_MAIN, CTX_LEN - OUT_MAIN)],
                                  out2_s.at[slot]),
        )

    def start_in(step, g):
        for cp in in_copies(step):
            cp.start(priority=g % 2)

    @pl.when(i == 0)
    def _prologue():
        for g in range(G):
            start_in(g, g)

    @pl.when(i + 1 < NITER)
    def _next_in():
        for g in range(G):
            start_in((i + 1) * G + g, g)

    for g in range(G):
        step = i * G + g
        for cp in in_copies(step):
            cp.wait()

        @pl.when(i >= 2)
        def _wait_prev_out():
            for cp in out_copies(step - NBUF):
                cp.wait()

        slot = lax.rem(step, NBUF)
        out_v[slot] = jnp.concatenate(
            [pre_v[slot], prm_v[slot], suf_v[slot]], axis=1)
        for cp in out_copies(step):
            cp.start(priority=g % 2)

    @pl.when(i == NITER - 1)
    def _drain():
        for j in range(NBUF):
            for cp in out_copies(NSTEP - 1 - j):
                cp.wait()


def kernel(new_prompt_tokens, token_prefix, token_suffix, tokenized_prompts):
    prompts = pl.pallas_call(
        _body,
        grid=(NITER,),
        in_specs=[
            pl.BlockSpec(memory_space=pl.ANY),
            pl.BlockSpec(memory_space=pl.ANY),
            pl.BlockSpec(memory_space=pl.ANY),
        ],
        out_specs=pl.BlockSpec(memory_space=pl.ANY),
        out_shape=jax.ShapeDtypeStruct((N_CLS, CTX_LEN, EMBED_DIM), jnp.float32),
        scratch_shapes=[
            pltpu.VMEM((NBUF, C, 1, EMBED_DIM), jnp.float32),
            pltpu.VMEM((NBUF, C, PROMPT_LEN, EMBED_DIM), jnp.float32),
            pltpu.VMEM((NBUF, C, SUF_LEN, EMBED_DIM), jnp.float32),
            pltpu.VMEM((NBUF, C, CTX_LEN, EMBED_DIM), jnp.float32),
            pltpu.SemaphoreType.DMA((NBUF,)),
            pltpu.SemaphoreType.DMA((NBUF,)),
            pltpu.SemaphoreType.DMA((NBUF,)),
            pltpu.SemaphoreType.DMA((NBUF,)),
            pltpu.SemaphoreType.DMA((NBUF,)),
            pltpu.SemaphoreType.DMA((NBUF,)),
        ],
        compiler_params=pltpu.CompilerParams(
            dimension_semantics=("arbitrary",),
        ),
    )(token_prefix, new_prompt_tokens, token_suffix)
    return (tokenized_prompts, prompts)
